# baseline (speedup 1.0000x reference)
import functools

import jax
import jax.numpy as jnp
from jax import lax
from jax.experimental import pallas as pl
from jax.experimental.pallas import tpu as pltpu
from jax.experimental.pallas import tpu_sc as plsc

VOCAB = 1000000
EMB = 64
PKW = EMB // 2
BATCH = 16384
CTX = 5
LANES = 16

_info = plsc.get_sparse_core_info()
_NC, _NS = _info.num_cores, _info.num_subcores
NW = _NC * _NS
BPW = BATCH // NW
CB = 128
NCHUNK = BPW // CB

_params = pltpu.CompilerParams(
    needs_layout_passes=False, use_tc_tiling_on_sc=False)
_mesh = plsc.VectorSubcoreMesh(core_axis_name="c", subcore_axis_name="s")


def _gd_body(t_hbm, c_hbm, t_tab, c_tab, out_hbm,
             tidx_v, cidx_v, cidx_m, trows_v, crows_v, out_v, sem):
    wid = lax.axis_index("s") * _NC + lax.axis_index("c")
    lane = lax.iota(jnp.int32, LANES)
    shift = jnp.full((LANES,), 16, jnp.int32)
    mask = jnp.full((LANES,), -65536, jnp.int32)

    for chunk in range(NCHUNK):
        base = wid * BPW + chunk * CB
        pltpu.sync_copy(t_hbm.at[pl.ds(base, CB)], tidx_v)
        pltpu.sync_copy(c_hbm.at[pl.ds(base * CTX, CB * CTX)], cidx_v)
        for j in range(CTX):
            for s in range(CB // LANES):
                cidx_m[j, pl.ds(s * LANES, LANES)] = (
                    cidx_v[pl.ds(j * CB + s * LANES, LANES)])

        copies = [pltpu.async_copy(t_tab.at[tidx_v], trows_v, sem)]
        for j in range(CTX):
            copies.append(
                pltpu.async_copy(c_tab.at[cidx_m.at[j]],
                                 crows_v.at[pl.ds(j * CB, CB)], sem))
        for cp in copies:
            cp.wait()

        for g in range(CB // LANES):
            wrow = g * LANES + lane
            crows = [wrow * CTX + c for c in range(CTX)]

            def body(e, accs, wrow=wrow, crows=crows):
                ev = jnp.full((LANES,), e, jnp.int32)
                wi = lax.bitcast_convert_type(
                    plsc.load_gather(trows_v, [wrow, ev]), jnp.int32)
                wlo = lax.bitcast_convert_type(wi << shift, jnp.float32)
                whi = lax.bitcast_convert_type(wi & mask, jnp.float32)
                out = []
                for c in range(CTX):
                    ci = lax.bitcast_convert_type(
                        plsc.load_gather(crows_v, [crows[c], ev]), jnp.int32)
                    clo = lax.bitcast_convert_type(ci << shift, jnp.float32)
                    chi = lax.bitcast_convert_type(ci & mask, jnp.float32)
                    out.append(accs[c] + wlo * clo + whi * chi)
                return tuple(out)

            accs = lax.fori_loop(
                0, PKW, body,
                tuple(jnp.zeros((LANES,), jnp.float32) for _ in range(CTX)))
            for c in range(CTX):
                plsc.store_scatter(out_v, [crows[c]], accs[c])

        pltpu.sync_copy(out_v, out_hbm.at[pl.ds(base * CTX, CB * CTX)])


def _pack(table):
    b = table.astype(jnp.bfloat16).reshape(VOCAB, PKW, 2)
    return lax.bitcast_convert_type(b, jnp.float32)


def kernel(target, context, target_table, context_table):
    t = target.reshape(BATCH).astype(jnp.int32)
    c = context.reshape(BATCH * CTX).astype(jnp.int32)

    gather_dot = functools.partial(
        pl.kernel,
        out_type=jax.ShapeDtypeStruct((BATCH * CTX,), jnp.float32),
        mesh=_mesh,
        compiler_params=_params,
        scratch_types=[
            pltpu.VMEM((CB,), jnp.int32),
            pltpu.VMEM((CB * CTX,), jnp.int32),
            pltpu.VMEM((CTX, CB), jnp.int32),
            pltpu.VMEM((CB, PKW), jnp.float32),
            pltpu.VMEM((CB * CTX, PKW), jnp.float32),
            pltpu.VMEM((CB * CTX,), jnp.float32),
            pltpu.SemaphoreType.DMA,
        ],
    )(_gd_body)
    dots = gather_dot(t, c, _pack(target_table), _pack(context_table))
    return dots.reshape(BATCH, CTX)
